# baseline (speedup 1.0000x reference)
import functools

import jax
import jax.numpy as jnp
from jax import lax
from jax.experimental import pallas as pl
from jax.experimental.pallas import tpu as pltpu
from jax.experimental.pallas import tpu_sc as plsc

N = 10000
D = 128
H = 128
OUT = 40
E = 320000

NC = 2
NS = 16
L = 16

NP = 10112
EP = 327680
CHUNK = 128
ECH = EP // (NC * NS) // CHUNK
BCH = 8
NBATCH = ECH // BCH
RPT = NP // NS
PADC = float(EP - E)

_mesh = plsc.VectorSubcoreMesh(core_axis_name="c", subcore_axis_name="s")


def _zero_rows(ref, nrows):
    z = jnp.zeros((L,), jnp.float32)

    def body(i, _):
        ref[i // 8, pl.ds((i % 8) * L, L)] = z
        return 0

    lax.fori_loop(0, nrows * 8, body, 0)


def _prop_body(table_hbm, src_hbm, dst_hbm, acc_hbm,
               src_v, dst_v, rows, shared_acc,
               gsem0, gsem1, ssem0, ssem1):
    gsems = (gsem0, gsem1)
    ssems = (ssem0, ssem1)
    zb = rows.at[0]
    c = lax.axis_index("c")
    s = lax.axis_index("s")
    w = s * NC + c

    _zero_rows(zb, CHUNK)
    for k in range(RPT // CHUNK):
        pltpu.sync_copy(
            zb, shared_acc.at[pl.ds(s * RPT + k * CHUNK, CHUNK)])
    rem = RPT - (RPT // CHUNK) * CHUNK
    if rem:
        pltpu.sync_copy(
            zb.at[pl.ds(0, rem)],
            shared_acc.at[pl.ds(s * RPT + (RPT // CHUNK) * CHUNK, rem)])

    plsc.subcore_barrier()

    def batch(b, _):
        pltpu.sync_copy(src_hbm.at[w, pl.ds(b * BCH, BCH)], src_v)
        pltpu.sync_copy(dst_hbm.at[w, pl.ds(b * BCH, BCH)], dst_v)

        hg = [None] * BCH
        hs = [None] * BCH
        hg[0] = pltpu.async_copy(table_hbm.at[src_v.at[0]], rows.at[0],
                                 gsems[0])
        for j in range(BCH):
            hg[j].wait()
            hs[j] = pltpu.async_copy(rows.at[j % 2],
                                     shared_acc.at[dst_v.at[j]],
                                     ssems[j % 2], add=True)
            if j >= 1:
                hs[j - 1].wait()
            if j + 1 < BCH:
                hg[j + 1] = pltpu.async_copy(
                    table_hbm.at[src_v.at[j + 1]],
                    rows.at[(j + 1) % 2], gsems[(j + 1) % 2])
        hs[BCH - 1].wait()
        return 0

    lax.fori_loop(0, NBATCH, batch, 0)

    plsc.subcore_barrier()
    nfull = RPT // CHUNK
    for k in range(nfull):
        pltpu.sync_copy(
            shared_acc.at[pl.ds(s * RPT + k * CHUNK, CHUNK)], zb)
        pltpu.sync_copy(
            zb, acc_hbm.at[c, pl.ds(s * RPT + k * CHUNK, CHUNK)])
    rem = RPT - nfull * CHUNK
    if rem:
        pltpu.sync_copy(
            shared_acc.at[pl.ds(s * RPT + nfull * CHUNK, rem)],
            zb.at[pl.ds(0, rem)])
        pltpu.sync_copy(
            zb.at[pl.ds(0, rem)],
            acc_hbm.at[c, pl.ds(s * RPT + nfull * CHUNK, rem)])


_prop_kernel = pl.kernel(
    _prop_body,
    out_type=jax.ShapeDtypeStruct((NC, NP, H), jnp.float32),
    mesh=_mesh,
    scratch_types=[
        pltpu.VMEM((BCH, CHUNK), jnp.int32),
        pltpu.VMEM((BCH, CHUNK), jnp.int32),
        pltpu.VMEM((2, CHUNK, H), jnp.float32),
        pltpu.VMEM_SHARED((NP, H), jnp.float32),
        pltpu.SemaphoreType.DMA,
        pltpu.SemaphoreType.DMA,
        pltpu.SemaphoreType.DMA,
        pltpu.SemaphoreType.DMA,
    ],
)


def _eye(n):
    r = lax.broadcasted_iota(jnp.int32, (n, n), 0)
    col = lax.broadcasted_iota(jnp.int32, (n, n), 1)
    return (r == col).astype(jnp.float32)


def _taylor_q(b, bt):
    s = b - bt
    s2 = jnp.dot(s, s, preferred_element_type=jnp.float32)
    s3 = jnp.dot(s2, s, preferred_element_type=jnp.float32)
    return _eye(H) + s + 0.5 * s2 + (1.0 / 6.0) * s3


def _row0_mask():
    m = (lax.broadcasted_iota(jnp.int32, (BR, 1), 0) == 0).astype(jnp.float32)
    return m * jnp.where(pl.program_id(0) == 0, 1.0, 0.0)


def _t1_body(x_ref, w0_ref, dacc_ref, dinv_ref, g_ref):
    dacc = dacc_ref[...]
    deg = (dacc[0] + dacc[1])[:, 0:1]
    deg = deg - _row0_mask() * PADC
    deg = jnp.maximum(deg, 1.0)
    dinv = lax.rsqrt(deg)
    g = jnp.dot(x_ref[...], w0_ref[...], preferred_element_type=jnp.float32)
    dinv_ref[...] = dinv
    g_ref[...] = g * dinv


def _layer_body(acc_ref, gprev_ref, dinv_ref, b_ref, bt_ref, hprev_ref,
                h_ref, g_ref, *, residual):
    dinv = dinv_ref[...]
    accsum = acc_ref[0] + acc_ref[1]
    accsum = accsum - (_row0_mask() * PADC) * gprev_ref[0:1, :]
    agg = accsum * dinv
    q = _taylor_q(b_ref[...], bt_ref[...])
    h = jnp.maximum(jnp.dot(agg, q, preferred_element_type=jnp.float32), 0.0)
    if residual:
        h = h + hprev_ref[...]
    h_ref[...] = h
    g_ref[...] = h * dinv


def _final_body(acc_ref, gprev_ref, dinv_ref, b_ref, bt_ref, hprev_ref,
                wout_ref, bout_ref, out_ref):
    dinv = dinv_ref[...]
    accsum = acc_ref[0] + acc_ref[1]
    accsum = accsum - (_row0_mask() * PADC) * gprev_ref[0:1, :]
    agg = accsum * dinv
    q = _taylor_q(b_ref[...], bt_ref[...])
    h = jnp.maximum(jnp.dot(agg, q, preferred_element_type=jnp.float32), 0.0)
    h = h + hprev_ref[...]
    out = jnp.dot(h, wout_ref[...], preferred_element_type=jnp.float32)
    out_ref[...] = out + bout_ref[...]


_f32 = jnp.float32
NB = 8
BR = NP // NB

_bs_rows = pl.BlockSpec((BR, H), lambda i: (i, 0))
_bs_dinv = pl.BlockSpec((BR, 1), lambda i: (i, 0))
_bs_acc = pl.BlockSpec((NC, BR, H), lambda i: (0, i, 0))
_bs_w = pl.BlockSpec((H, H), lambda i: (0, 0))

_t1_call = pl.pallas_call(
    _t1_body,
    grid=(NB,),
    in_specs=[_bs_rows, _bs_w, _bs_acc],
    out_specs=[_bs_dinv, _bs_rows],
    out_shape=[jax.ShapeDtypeStruct((NP, 1), _f32),
               jax.ShapeDtypeStruct((NP, H), _f32)],
)

_layer_specs = dict(
    grid=(NB,),
    in_specs=[_bs_acc, _bs_rows, _bs_dinv, _bs_w, _bs_w, _bs_rows],
    out_specs=[_bs_rows, _bs_rows],
    out_shape=[jax.ShapeDtypeStruct((NP, H), _f32),
               jax.ShapeDtypeStruct((NP, H), _f32)],
)

_layer_call = pl.pallas_call(
    functools.partial(_layer_body, residual=True), **_layer_specs)

_layer0_call = pl.pallas_call(
    functools.partial(_layer_body, residual=False), **_layer_specs)

_final_call = pl.pallas_call(
    _final_body,
    grid=(NB,),
    in_specs=[_bs_acc, _bs_rows, _bs_dinv, _bs_w, _bs_w, _bs_rows,
              pl.BlockSpec((H, OUT), lambda i: (0, 0)),
              pl.BlockSpec((1, OUT), lambda i: (0, 0))],
    out_specs=pl.BlockSpec((BR, OUT), lambda i: (i, 0)),
    out_shape=jax.ShapeDtypeStruct((NP, OUT), _f32),
)


def kernel(x, edge_index, W0, B0, B1, B2, W_out, b_out):
    src = edge_index[0]
    dst = edge_index[1]
    pad = jnp.zeros((EP - E,), jnp.int32)
    src_p = jnp.concatenate([src, pad]).reshape(NC * NS, ECH, CHUNK)
    dst_p = jnp.concatenate([dst, pad]).reshape(NC * NS, ECH, CHUNK)

    ones_t = jnp.ones((NP, H), jnp.float32)

    deg_acc = _prop_kernel(ones_t, src_p, dst_p)
    x_p = jnp.pad(x, ((0, NP - N), (0, 0)))
    dinv, g0 = _t1_call(x_p, W0, deg_acc)

    acc0 = _prop_kernel(g0, src_p, dst_p)
    h1, g1 = _layer0_call(acc0, g0, dinv, B0, B0.T, x_p)

    acc1 = _prop_kernel(g1, src_p, dst_p)
    h2, g2 = _layer_call(acc1, g1, dinv, B1, B1.T, h1)

    acc2 = _prop_kernel(g2, src_p, dst_p)
    out = _final_call(acc2, g2, dinv, B2, B2.T, h2,
                      W_out, b_out.reshape(1, OUT))
    return out[:N]


# speedup vs baseline: 1.0005x; 1.0005x over previous
import functools

import jax
import jax.numpy as jnp
from jax import lax
from jax.experimental import pallas as pl
from jax.experimental.pallas import tpu as pltpu
from jax.experimental.pallas import tpu_sc as plsc

N = 10000
D = 128
H = 128
OUT = 40
E = 320000

NC = 2
NS = 16
L = 16

NP = 10112
EP = 327680
CHUNK = 128
ECH = EP // (NC * NS) // CHUNK
BCH = 8
NBATCH = ECH // BCH
RPT = NP // NS
PADC = float(EP - E)

_mesh = plsc.VectorSubcoreMesh(core_axis_name="c", subcore_axis_name="s")


def _zero_rows(ref, nrows):
    z = jnp.zeros((L,), jnp.float32)

    def body(i, _):
        ref[i // 8, pl.ds((i % 8) * L, L)] = z
        return 0

    lax.fori_loop(0, nrows * 8, body, 0)


def _prop_body(table_hbm, src_hbm, dst_hbm, acc_hbm,
               src_v, dst_v, rows, shared_acc,
               gsem0, gsem1):
    gsems = (gsem0, gsem1)
    zb = rows.at[0]
    c = lax.axis_index("c")
    s = lax.axis_index("s")
    w = s * NC + c

    _zero_rows(zb, CHUNK)
    for k in range(RPT // CHUNK):
        pltpu.sync_copy(
            zb, shared_acc.at[pl.ds(s * RPT + k * CHUNK, CHUNK)])
    rem = RPT - (RPT // CHUNK) * CHUNK
    if rem:
        pltpu.sync_copy(
            zb.at[pl.ds(0, rem)],
            shared_acc.at[pl.ds(s * RPT + (RPT // CHUNK) * CHUNK, rem)])

    plsc.subcore_barrier()

    def batch(b, _):
        pltpu.sync_copy(src_hbm.at[w, pl.ds(b * BCH, BCH)], src_v)
        pltpu.sync_copy(dst_hbm.at[w, pl.ds(b * BCH, BCH)], dst_v)

        hg = [None] * BCH
        hg[0] = pltpu.async_copy(table_hbm.at[src_v.at[0]], rows.at[0],
                                 gsems[0])
        for j in range(BCH):
            hg[j].wait()
            if j + 1 < BCH:
                hg[j + 1] = pltpu.async_copy(
                    table_hbm.at[src_v.at[j + 1]],
                    rows.at[(j + 1) % 2], gsems[(j + 1) % 2])
            pltpu.sync_copy(rows.at[j % 2],
                            shared_acc.at[dst_v.at[j]], add=True)
        return 0

    lax.fori_loop(0, NBATCH, batch, 0)

    plsc.subcore_barrier()
    nfull = RPT // CHUNK
    for k in range(nfull):
        pltpu.sync_copy(
            shared_acc.at[pl.ds(s * RPT + k * CHUNK, CHUNK)], zb)
        pltpu.sync_copy(
            zb, acc_hbm.at[c, pl.ds(s * RPT + k * CHUNK, CHUNK)])
    rem = RPT - nfull * CHUNK
    if rem:
        pltpu.sync_copy(
            shared_acc.at[pl.ds(s * RPT + nfull * CHUNK, rem)],
            zb.at[pl.ds(0, rem)])
        pltpu.sync_copy(
            zb.at[pl.ds(0, rem)],
            acc_hbm.at[c, pl.ds(s * RPT + nfull * CHUNK, rem)])


_prop_kernel = pl.kernel(
    _prop_body,
    out_type=jax.ShapeDtypeStruct((NC, NP, H), jnp.float32),
    mesh=_mesh,
    scratch_types=[
        pltpu.VMEM((BCH, CHUNK), jnp.int32),
        pltpu.VMEM((BCH, CHUNK), jnp.int32),
        pltpu.VMEM((2, CHUNK, H), jnp.float32),
        pltpu.VMEM_SHARED((NP, H), jnp.float32),
        pltpu.SemaphoreType.DMA,
        pltpu.SemaphoreType.DMA,
    ],
)


def _eye(n):
    r = lax.broadcasted_iota(jnp.int32, (n, n), 0)
    col = lax.broadcasted_iota(jnp.int32, (n, n), 1)
    return (r == col).astype(jnp.float32)


def _taylor_q(b, bt):
    s = b - bt
    s2 = jnp.dot(s, s, preferred_element_type=jnp.float32)
    s3 = jnp.dot(s2, s, preferred_element_type=jnp.float32)
    return _eye(H) + s + 0.5 * s2 + (1.0 / 6.0) * s3


def _row0_mask():
    m = (lax.broadcasted_iota(jnp.int32, (BR, 1), 0) == 0).astype(jnp.float32)
    return m * jnp.where(pl.program_id(0) == 0, 1.0, 0.0)


def _t1_body(x_ref, w0_ref, dacc_ref, dinv_ref, g_ref):
    dacc = dacc_ref[...]
    deg = (dacc[0] + dacc[1])[:, 0:1]
    deg = deg - _row0_mask() * PADC
    deg = jnp.maximum(deg, 1.0)
    dinv = lax.rsqrt(deg)
    g = jnp.dot(x_ref[...], w0_ref[...], preferred_element_type=jnp.float32)
    dinv_ref[...] = dinv
    g_ref[...] = g * dinv


def _layer_body(acc_ref, gprev_ref, dinv_ref, b_ref, bt_ref, hprev_ref,
                h_ref, g_ref, *, residual):
    dinv = dinv_ref[...]
    accsum = acc_ref[0] + acc_ref[1]
    accsum = accsum - (_row0_mask() * PADC) * gprev_ref[0:1, :]
    agg = accsum * dinv
    q = _taylor_q(b_ref[...], bt_ref[...])
    h = jnp.maximum(jnp.dot(agg, q, preferred_element_type=jnp.float32), 0.0)
    if residual:
        h = h + hprev_ref[...]
    h_ref[...] = h
    g_ref[...] = h * dinv


def _final_body(acc_ref, gprev_ref, dinv_ref, b_ref, bt_ref, hprev_ref,
                wout_ref, bout_ref, out_ref):
    dinv = dinv_ref[...]
    accsum = acc_ref[0] + acc_ref[1]
    accsum = accsum - (_row0_mask() * PADC) * gprev_ref[0:1, :]
    agg = accsum * dinv
    q = _taylor_q(b_ref[...], bt_ref[...])
    h = jnp.maximum(jnp.dot(agg, q, preferred_element_type=jnp.float32), 0.0)
    h = h + hprev_ref[...]
    out = jnp.dot(h, wout_ref[...], preferred_element_type=jnp.float32)
    out_ref[...] = out + bout_ref[...]


_f32 = jnp.float32
NB = 8
BR = NP // NB

_bs_rows = pl.BlockSpec((BR, H), lambda i: (i, 0))
_bs_dinv = pl.BlockSpec((BR, 1), lambda i: (i, 0))
_bs_acc = pl.BlockSpec((NC, BR, H), lambda i: (0, i, 0))
_bs_w = pl.BlockSpec((H, H), lambda i: (0, 0))

_t1_call = pl.pallas_call(
    _t1_body,
    grid=(NB,),
    in_specs=[_bs_rows, _bs_w, _bs_acc],
    out_specs=[_bs_dinv, _bs_rows],
    out_shape=[jax.ShapeDtypeStruct((NP, 1), _f32),
               jax.ShapeDtypeStruct((NP, H), _f32)],
)

_layer_specs = dict(
    grid=(NB,),
    in_specs=[_bs_acc, _bs_rows, _bs_dinv, _bs_w, _bs_w, _bs_rows],
    out_specs=[_bs_rows, _bs_rows],
    out_shape=[jax.ShapeDtypeStruct((NP, H), _f32),
               jax.ShapeDtypeStruct((NP, H), _f32)],
)

_layer_call = pl.pallas_call(
    functools.partial(_layer_body, residual=True), **_layer_specs)

_layer0_call = pl.pallas_call(
    functools.partial(_layer_body, residual=False), **_layer_specs)

_final_call = pl.pallas_call(
    _final_body,
    grid=(NB,),
    in_specs=[_bs_acc, _bs_rows, _bs_dinv, _bs_w, _bs_w, _bs_rows,
              pl.BlockSpec((H, OUT), lambda i: (0, 0)),
              pl.BlockSpec((1, OUT), lambda i: (0, 0))],
    out_specs=pl.BlockSpec((BR, OUT), lambda i: (i, 0)),
    out_shape=jax.ShapeDtypeStruct((NP, OUT), _f32),
)


def kernel(x, edge_index, W0, B0, B1, B2, W_out, b_out):
    src = edge_index[0]
    dst = edge_index[1]
    pad = jnp.zeros((EP - E,), jnp.int32)
    src_p = jnp.concatenate([src, pad]).reshape(NC * NS, ECH, CHUNK)
    dst_p = jnp.concatenate([dst, pad]).reshape(NC * NS, ECH, CHUNK)

    ones_t = jnp.ones((NP, H), jnp.float32)

    deg_acc = _prop_kernel(ones_t, src_p, dst_p)
    x_p = jnp.pad(x, ((0, NP - N), (0, 0)))
    dinv, g0 = _t1_call(x_p, W0, deg_acc)

    acc0 = _prop_kernel(g0, src_p, dst_p)
    h1, g1 = _layer0_call(acc0, g0, dinv, B0, B0.T, x_p)

    acc1 = _prop_kernel(g1, src_p, dst_p)
    h2, g2 = _layer_call(acc1, g1, dinv, B1, B1.T, h1)

    acc2 = _prop_kernel(g2, src_p, dst_p)
    out = _final_call(acc2, g2, dinv, B2, B2.T, h2,
                      W_out, b_out.reshape(1, OUT))
    return out[:N]


# speedup vs baseline: 1.0811x; 1.0806x over previous
import functools

import jax
import jax.numpy as jnp
from jax import lax
from jax.experimental import pallas as pl
from jax.experimental.pallas import tpu as pltpu
from jax.experimental.pallas import tpu_sc as plsc

N = 10000
D = 128
H = 128
OUT = 40
E = 320000

NC = 2
NS = 16
L = 16

NP = 10112
EP = 327680
CHUNK = 128
ECH = EP // (NC * NS) // CHUNK
BCH = 8
NBATCH = ECH // BCH
RPT = NP // NS

_mesh = plsc.VectorSubcoreMesh(core_axis_name="c", subcore_axis_name="s")


def _zero_rows(ref, nrows):
    z = jnp.zeros((L,), jnp.float32)

    def body(i, _):
        ref[i // 8, pl.ds((i % 8) * L, L)] = z
        return 0

    lax.fori_loop(0, nrows * 8, body, 0)


def _prop_body(table_hbm, src_hbm, dst_hbm, acc_hbm,
               src_v, dst_v, rows, shared_acc,
               gsem0, gsem1):
    gsems = (gsem0, gsem1)
    zb = rows.at[0]
    c = lax.axis_index("c")
    s = lax.axis_index("s")
    w = s * NC + c

    _zero_rows(zb, CHUNK)
    for k in range(RPT // CHUNK):
        pltpu.sync_copy(
            zb, shared_acc.at[pl.ds(s * RPT + k * CHUNK, CHUNK)])
    rem = RPT - (RPT // CHUNK) * CHUNK
    if rem:
        pltpu.sync_copy(
            zb.at[pl.ds(0, rem)],
            shared_acc.at[pl.ds(s * RPT + (RPT // CHUNK) * CHUNK, rem)])

    plsc.subcore_barrier()

    def batch(b, _):
        pltpu.sync_copy(src_hbm.at[w, pl.ds(b * BCH, BCH)], src_v)
        pltpu.sync_copy(dst_hbm.at[w, pl.ds(b * BCH, BCH)], dst_v)

        hg = [None] * BCH
        hg[0] = pltpu.async_copy(table_hbm.at[src_v.at[0]], rows.at[0],
                                 gsems[0])
        for j in range(BCH):
            hg[j].wait()
            if j + 1 < BCH:
                hg[j + 1] = pltpu.async_copy(
                    table_hbm.at[src_v.at[j + 1]],
                    rows.at[(j + 1) % 2], gsems[(j + 1) % 2])
            pltpu.sync_copy(rows.at[j % 2],
                            shared_acc.at[dst_v.at[j]], add=True)
        return 0

    lax.fori_loop(0, NBATCH, batch, 0)

    plsc.subcore_barrier()
    nfull = RPT // CHUNK
    for k in range(nfull):
        pltpu.sync_copy(
            shared_acc.at[pl.ds(s * RPT + k * CHUNK, CHUNK)], zb)
        pltpu.sync_copy(
            zb, acc_hbm.at[c, pl.ds(s * RPT + k * CHUNK, CHUNK)])
    rem = RPT - nfull * CHUNK
    if rem:
        pltpu.sync_copy(
            shared_acc.at[pl.ds(s * RPT + nfull * CHUNK, rem)],
            zb.at[pl.ds(0, rem)])
        pltpu.sync_copy(
            zb.at[pl.ds(0, rem)],
            acc_hbm.at[c, pl.ds(s * RPT + nfull * CHUNK, rem)])


_prop_kernel = pl.kernel(
    _prop_body,
    out_type=jax.ShapeDtypeStruct((NC, NP, H), jnp.float32),
    mesh=_mesh,
    scratch_types=[
        pltpu.VMEM((BCH, CHUNK), jnp.int32),
        pltpu.VMEM((BCH, CHUNK), jnp.int32),
        pltpu.VMEM((2, CHUNK, H), jnp.float32),
        pltpu.VMEM_SHARED((NP, H), jnp.float32),
        pltpu.SemaphoreType.DMA,
        pltpu.SemaphoreType.DMA,
    ],
)


def _eye(n):
    r = lax.broadcasted_iota(jnp.int32, (n, n), 0)
    col = lax.broadcasted_iota(jnp.int32, (n, n), 1)
    return (r == col).astype(jnp.float32)


def _taylor_q(b, bt):
    s = b - bt
    s2 = jnp.dot(s, s, preferred_element_type=jnp.float32)
    s3 = jnp.dot(s2, s, preferred_element_type=jnp.float32)
    return _eye(H) + s + 0.5 * s2 + (1.0 / 6.0) * s3


def _t1_body(x_ref, w0_ref, dacc_ref, dinv_ref, g_ref):
    dacc = dacc_ref[...]
    deg = (dacc[0] + dacc[1])[:, 0:1]
    deg = jnp.maximum(deg, 1.0)
    dinv = lax.rsqrt(deg)
    g = jnp.dot(x_ref[...], w0_ref[...], preferred_element_type=jnp.float32)
    dinv_ref[...] = dinv
    g_ref[...] = g * dinv


def _layer_body(acc_ref, dinv_ref, b_ref, bt_ref, hprev_ref,
                h_ref, g_ref, *, residual):
    dinv = dinv_ref[...]
    agg = (acc_ref[0] + acc_ref[1]) * dinv
    q = _taylor_q(b_ref[...], bt_ref[...])
    h = jnp.maximum(jnp.dot(agg, q, preferred_element_type=jnp.float32), 0.0)
    if residual:
        h = h + hprev_ref[...]
    h_ref[...] = h
    g_ref[...] = h * dinv


def _final_body(acc_ref, dinv_ref, b_ref, bt_ref, hprev_ref,
                wout_ref, bout_ref, out_ref):
    dinv = dinv_ref[...]
    agg = (acc_ref[0] + acc_ref[1]) * dinv
    q = _taylor_q(b_ref[...], bt_ref[...])
    h = jnp.maximum(jnp.dot(agg, q, preferred_element_type=jnp.float32), 0.0)
    h = h + hprev_ref[...]
    out = jnp.dot(h, wout_ref[...], preferred_element_type=jnp.float32)
    out_ref[...] = out + bout_ref[...]


_f32 = jnp.float32
NB = 8
BR = NP // NB

_bs_rows = pl.BlockSpec((BR, H), lambda i: (i, 0))
_bs_dinv = pl.BlockSpec((BR, 1), lambda i: (i, 0))
_bs_acc = pl.BlockSpec((NC, BR, H), lambda i: (0, i, 0))
_bs_w = pl.BlockSpec((H, H), lambda i: (0, 0))

_t1_call = pl.pallas_call(
    _t1_body,
    grid=(NB,),
    in_specs=[_bs_rows, _bs_w, _bs_acc],
    out_specs=[_bs_dinv, _bs_rows],
    out_shape=[jax.ShapeDtypeStruct((NP, 1), _f32),
               jax.ShapeDtypeStruct((NP, H), _f32)],
)

_layer_specs = dict(
    grid=(NB,),
    in_specs=[_bs_acc, _bs_dinv, _bs_w, _bs_w, _bs_rows],
    out_specs=[_bs_rows, _bs_rows],
    out_shape=[jax.ShapeDtypeStruct((NP, H), _f32),
               jax.ShapeDtypeStruct((NP, H), _f32)],
)

_layer_call = pl.pallas_call(
    functools.partial(_layer_body, residual=True), **_layer_specs)

_layer0_call = pl.pallas_call(
    functools.partial(_layer_body, residual=False), **_layer_specs)

_final_call = pl.pallas_call(
    _final_body,
    grid=(NB,),
    in_specs=[_bs_acc, _bs_dinv, _bs_w, _bs_w, _bs_rows,
              pl.BlockSpec((H, OUT), lambda i: (0, 0)),
              pl.BlockSpec((1, OUT), lambda i: (0, 0))],
    out_specs=pl.BlockSpec((BR, OUT), lambda i: (i, 0)),
    out_shape=jax.ShapeDtypeStruct((NP, OUT), _f32),
)


def kernel(x, edge_index, W0, B0, B1, B2, W_out, b_out):
    src = edge_index[0]
    dst = edge_index[1]
    pad = jnp.full((EP - E,), N, jnp.int32)
    src_p = jnp.concatenate([src, pad]).reshape(NC * NS, ECH, CHUNK)
    dst_p = jnp.concatenate([dst, pad]).reshape(NC * NS, ECH, CHUNK)

    ones_t = jnp.ones((NP, H), jnp.float32)

    deg_acc = _prop_kernel(ones_t, src_p, dst_p)
    x_p = jnp.pad(x, ((0, NP - N), (0, 0)))
    dinv, g0 = _t1_call(x_p, W0, deg_acc)

    acc0 = _prop_kernel(g0, src_p, dst_p)
    h1, g1 = _layer0_call(acc0, dinv, B0, B0.T, x_p)

    acc1 = _prop_kernel(g1, src_p, dst_p)
    h2, g2 = _layer_call(acc1, dinv, B1, B1.T, h1)

    acc2 = _prop_kernel(g2, src_p, dst_p)
    out = _final_call(acc2, dinv, B2, B2.T, h2,
                      W_out, b_out.reshape(1, OUT))
    return out[:N]
